# baseline (speedup 1.0000x reference)
import functools

import jax
import jax.numpy as jnp
from jax.experimental import pallas as pl
from jax.experimental.pallas import tpu as pltpu

_N = 10000
_BM = 400


def _mm_kernel(a_ref, w_ref, o_ref):
    a = a_ref[...].astype(jnp.bfloat16)
    w = w_ref[...].astype(jnp.bfloat16)
    o_ref[...] = jnp.dot(a, w, preferred_element_type=jnp.float32).astype(
        jnp.bfloat16)


def _agg_kernel(adj_ref, s_ref, b_ref, o_ref, *, relu):
    adj_blk = adj_ref[...].astype(jnp.bfloat16)
    acc = jnp.dot(adj_blk, s_ref[...], preferred_element_type=jnp.float32)
    acc = acc + b_ref[...]
    if relu:
        acc = jnp.maximum(acc, 0.0)
        o_ref[...] = acc.astype(jnp.bfloat16)
    else:
        o_ref[...] = acc


def _lsm_kernel(h_ref, o_ref):
    h = h_ref[...]
    m = jnp.max(h, axis=0, keepdims=True)
    lse = jnp.log(jnp.sum(jnp.exp(h - m), axis=0, keepdims=True)) + m
    o_ref[...] = h - lse


def _mm(a, w, out_dtype=jnp.bfloat16):
    m, k = a.shape
    _, n = w.shape
    return pl.pallas_call(
        _mm_kernel,
        out_shape=jax.ShapeDtypeStruct((m, n), out_dtype),
    )(a, w)


def _agg(adj, s, b, relu, out_dtype):
    n_rows = adj.shape[0]
    f = s.shape[1]
    grid = (n_rows // _BM,)
    return pl.pallas_call(
        functools.partial(_agg_kernel, relu=relu),
        grid=grid,
        in_specs=[
            pl.BlockSpec((_BM, _N), lambda i: (i, 0)),
            pl.BlockSpec((_N, f), lambda i: (0, 0)),
            pl.BlockSpec((1, f), lambda i: (0, 0)),
        ],
        out_specs=pl.BlockSpec((_BM, f), lambda i: (i, 0)),
        out_shape=jax.ShapeDtypeStruct((n_rows, f), out_dtype),
        compiler_params=pltpu.CompilerParams(
            dimension_semantics=("parallel",)),
    )(adj, s, b)


def _lsm(h):
    return pl.pallas_call(
        _lsm_kernel,
        out_shape=jax.ShapeDtypeStruct(h.shape, jnp.float32),
    )(h)


def kernel(x, adj, W0, b0, W1, b1):
    x2d = x.reshape(_N, x.shape[-1])
    s1 = _mm(x2d, W0)
    h1 = _agg(adj, s1, b0.reshape(1, -1), True, jnp.bfloat16)
    s2 = _mm(h1, W1)
    h2 = _agg(adj, s2, b1.reshape(1, -1), False, jnp.float32)
    out = _lsm(h2)
    return out.reshape(1, _N, -1)
